# Initial kernel scaffold; baseline (speedup 1.0000x reference)
import jax
import jax.numpy as jnp
from jax.experimental import pallas as pl


def kernel(x, coeffs, bias):
    raise NotImplementedError("write your pallas kernel here")


# speedup vs baseline: 19.2724x; 19.2724x over previous
import functools
import jax
import jax.numpy as jnp
from jax.experimental import pallas as pl
from jax.experimental.pallas import tpu as pltpu

X_MIN = -3.0
X_MAX = 3.0
KN = 64
IN = 64
ON = 256
H = (X_MAX - X_MIN) / (KN - 1)


def _prep_body(ct_ref, g_ref):
    y = ct_ref[...]
    delta = (y[1:, :] - y[:-1, :]) * (1.0 / H)
    d0 = (3.0 * delta[0:1, :] - delta[1:2, :]) * 0.5
    dN = (3.0 * delta[KN - 2:KN - 1, :] - delta[KN - 3:KN - 2, :]) * 0.5

    def fix_end(d_end, delta0, delta1):
        d_end = jnp.where(d_end * delta0 <= 0.0, 0.0, d_end)
        bad = (delta0 * delta1 < 0.0) & (jnp.abs(d_end) > 3.0 * jnp.abs(delta0))
        return jnp.where(bad, 3.0 * delta0, d_end)

    d0 = fix_end(d0, delta[0:1, :], delta[1:2, :])
    dN = fix_end(dN, delta[KN - 2:KN - 1, :], delta[KN - 3:KN - 2, :])
    dp = delta[:-1, :]
    dn = delta[1:, :]
    same = dp * dn > 0.0
    dmid = jnp.where(same, 2.0 * dp * dn / (dp + dn + 1e-12), 0.0)
    d = jnp.concatenate([d0, dmid, dN], axis=0)
    g_ref[0:KN, :] = y
    g_ref[KN:2 * KN, :] = H * d


def _main_body(x_ref, g_ref, b_ref, o_ref):
    x = x_ref[...]
    t = (x - X_MIN) * (1.0 / H)
    idx = jnp.clip(jnp.floor(t), 0.0, float(KN - 2))
    u = t - idx
    u2 = u * u
    u3 = u2 * u
    h00 = 2.0 * u3 - 3.0 * u2 + 1.0
    h10 = u3 - 2.0 * u2 + u
    h01 = 3.0 * u2 - 2.0 * u3
    h11 = u3 - u2
    left = t < 0.0
    right = t > float(KN - 1)
    wy0 = jnp.where(left, 1.0, jnp.where(right, 0.0, h00))
    wd0 = jnp.where(left, t, jnp.where(right, 0.0, h10))
    wy1 = jnp.where(left, 0.0, jnp.where(right, 1.0, h01))
    wd1 = jnp.where(left, 0.0, jnp.where(right, u - 1.0, h11))

    bt = x.shape[0]
    idx3 = idx.astype(jnp.int32)[:, :, None]
    kk = jax.lax.broadcasted_iota(jnp.int32, (1, 1, 2 * KN), 2)
    kmod = jnp.where(kk < KN, kk, kk - KN)
    isy = kk < KN
    wlo = jnp.where(isy, wy0[:, :, None], wd0[:, :, None])
    whi = jnp.where(isy, wy1[:, :, None], wd1[:, :, None])
    s = jnp.where(kmod == idx3, wlo,
                  jnp.where(kmod == idx3 + 1, whi, 0.0))
    s2 = s.reshape(bt, IN * 2 * KN)
    acc = jax.lax.dot_general(
        s2, g_ref[...], (((1,), (0,)), ((), ())),
        preferred_element_type=jnp.float32,
        precision=jax.lax.Precision.HIGHEST)
    o_ref[...] = acc + b_ref[...]


@jax.jit
def _run(x, coeffs, bias):
    ct = jnp.transpose(coeffs, (1, 2, 0)).reshape(IN * KN, ON)
    g = pl.pallas_call(
        _prep_body,
        grid=(IN,),
        in_specs=[pl.BlockSpec((KN, ON), lambda i: (i, 0))],
        out_specs=pl.BlockSpec((2 * KN, ON), lambda i: (i, 0)),
        out_shape=jax.ShapeDtypeStruct((IN * 2 * KN, ON), jnp.float32),
    )(ct)

    BT = 256
    B = x.shape[0]
    out = pl.pallas_call(
        _main_body,
        grid=(B // BT,),
        in_specs=[
            pl.BlockSpec((BT, IN), lambda i: (i, 0)),
            pl.BlockSpec((IN * 2 * KN, ON), lambda i: (0, 0)),
            pl.BlockSpec((1, ON), lambda i: (0, 0)),
        ],
        out_specs=pl.BlockSpec((BT, ON), lambda i: (i, 0)),
        out_shape=jax.ShapeDtypeStruct((B, ON), jnp.float32),
    )(x, g, bias.reshape(1, ON))
    return out


def kernel(x, coeffs, bias):
    return _run(x, coeffs, bias)
